# baseline (speedup 1.0000x reference)
import jax
import jax.numpy as jnp
from jax import lax
from jax.experimental import pallas as pl
from jax.experimental.pallas import tpu as pltpu
from jax.experimental.pallas import tpu_sc as plsc

_B, _H, _W, _C = 4, 112, 112, 96
_N = _B * _H * _W * _C
_OUT = _N * 4

_NSC = 2
_NT = 16
_L = 16

_NCHUNK = 12
_CH = _OUT // _NCHUNK
_CPS = _NCHUNK // _NSC
_DUMMY = 32768
_ACC = _CH + _DUMMY

_SHARE = _N // _NT
_WIN = 4704
_NWIN = _SHARE // _WIN
_VSTEP = _WIN // _L
_UNROLL = 6
_NB = 3

_TSLICE = _CH // _NT


def _body(idx_hbm, upd_hbm, zero_hbm, out_hbm,
          i0, i1, i2, v0, v1, v2, acc, lisem, lvsem, scsem, zsem):
    cid = lax.axis_index("c")
    sid = lax.axis_index("s")
    in_base = sid * _SHARE
    ib = (i0, i1, i2)
    vb = (v0, v1, v2)

    def _issue_load(w, b):
        base = in_base + w * _WIN
        pltpu.async_copy(idx_hbm.at[pl.ds(base, _WIN)], ib[b], lisem.at[b])
        pltpu.async_copy(upd_hbm.at[pl.ds(base, _WIN)], vb[b], lvsem.at[b])

    def _wait_load_idx(w, b):
        base = in_base + w * _WIN
        pltpu.make_async_copy(idx_hbm.at[pl.ds(base, _WIN)], ib[b],
                              lisem.at[b]).wait()

    def _wait_load_val(w, b):
        base = in_base + w * _WIN
        pltpu.make_async_copy(upd_hbm.at[pl.ds(base, _WIN)], vb[b],
                              lvsem.at[b]).wait()

    def _filter(b, lo):
        @plsc.parallel_loop(0, _VSTEP, step=1, unroll=_UNROLL)
        def _vec(j):
            sl = pl.ds(j * _L, _L)
            x = ib[b][sl]
            u = plsc.bitcast(x - lo, jnp.uint32)
            d = plsc.bitcast((x & (_DUMMY - 1)) + _CH, jnp.uint32)
            ib[b][sl] = plsc.bitcast(jnp.minimum(u, d), jnp.int32)

    def _issue_scatter(b):
        pltpu.async_copy(vb[b], acc.at[ib[b]], scsem.at[b], add=True)

    def _wait_scatter(b):
        pltpu.make_async_copy(vb[b], acc.at[ib[b]], scsem.at[b]).wait()

    def _chunk(k, carry):
        lo = (cid * _CPS + k) * _CH
        zbase = pl.multiple_of(sid * _TSLICE, 8)

        pltpu.async_copy(zero_hbm.at[pl.ds(zbase, _TSLICE)],
                         acc.at[pl.ds(zbase, _TSLICE)], zsem)
        _issue_load(0, 0)
        pltpu.make_async_copy(zero_hbm.at[pl.ds(zbase, _TSLICE)],
                              acc.at[pl.ds(zbase, _TSLICE)], zsem).wait()
        plsc.subcore_barrier()

        def _group(g, c2):
            for b in range(_NB):
                w = g * _NB + b
                pf = (b + 1) % _NB
                if b < _NB - 1:
                    @pl.when(g > 0)
                    def _():
                        _wait_scatter(pf)
                else:
                    _wait_scatter(pf)
                _issue_load(w + 1, pf)
                _wait_load_idx(w, b)
                _filter(b, lo)
                _wait_load_val(w, b)
                _issue_scatter(b)
            return c2
        lax.fori_loop(0, (_NWIN - 1) // _NB, _group, 0)

        wlast = _NWIN - 1
        _wait_scatter(1)
        _wait_load_idx(wlast, 0)
        _filter(0, lo)
        _wait_load_val(wlast, 0)
        _issue_scatter(0)
        _wait_scatter(2)
        _wait_scatter(0)
        plsc.subcore_barrier()

        off = pl.multiple_of(lo + sid * _TSLICE, 8)
        pltpu.sync_copy(acc.at[pl.ds(zbase, _TSLICE)],
                        out_hbm.at[pl.ds(off, _TSLICE)])
        return carry

    lax.fori_loop(0, _CPS, _chunk, 0)


def kernel(updates, mask):
    idx = mask.reshape(-1)
    upd = updates.reshape(-1)
    f = pl.kernel(
        _body,
        out_type=jax.ShapeDtypeStruct((_OUT,), jnp.float32),
        mesh=plsc.VectorSubcoreMesh(core_axis_name="c", subcore_axis_name="s"),
        scratch_types=[
            pltpu.VMEM((_WIN,), jnp.int32),
            pltpu.VMEM((_WIN,), jnp.int32),
            pltpu.VMEM((_WIN,), jnp.int32),
            pltpu.VMEM((_WIN,), jnp.float32),
            pltpu.VMEM((_WIN,), jnp.float32),
            pltpu.VMEM((_WIN,), jnp.float32),
            pltpu.VMEM_SHARED((_ACC,), jnp.float32),
            pltpu.SemaphoreType.DMA((_NB,)),
            pltpu.SemaphoreType.DMA((_NB,)),
            pltpu.SemaphoreType.DMA((_NB,)),
            pltpu.SemaphoreType.DMA,
        ],
    )
    zeros = jnp.zeros((_CH,), jnp.float32)
    out = f(idx, upd, zeros)
    return out.reshape(_B, _H * 2, _W * 2, _C)


# speedup vs baseline: 1.0441x; 1.0441x over previous
import jax
import jax.numpy as jnp
from jax import lax
from jax.experimental import pallas as pl
from jax.experimental.pallas import tpu as pltpu
from jax.experimental.pallas import tpu_sc as plsc

_B, _H, _W, _C = 4, 112, 112, 96
_N = _B * _H * _W * _C
_OUT = _N * 4

_NSC = 2
_NT = 16
_L = 16

_NCHUNK = 10
_CH = 1927168
_CPS = _NCHUNK // _NSC
_DUMMY = 4096
_ACC = _CH + _DUMMY

_SHARE = _N // _NT
_WIN = 1568
_NWIN = _SHARE // _WIN
_VSTEP = _WIN // _L
_UNROLL = 7
_NB = 3

_TSLICE = _CH // _NT
_TCLIP = _TSLICE - (_NCHUNK * _CH - _OUT)


def _body(idx_hbm, upd_hbm, zero_hbm, out_hbm,
          i0, i1, i2, v0, v1, v2, acc, lisem, lvsem, scsem, zsem):
    cid = lax.axis_index("c")
    sid = lax.axis_index("s")
    in_base = sid * _SHARE
    ib = (i0, i1, i2)
    vb = (v0, v1, v2)

    def _issue_load(w, b):
        base = in_base + w * _WIN
        pltpu.async_copy(idx_hbm.at[pl.ds(base, _WIN)], ib[b], lisem.at[b])
        pltpu.async_copy(upd_hbm.at[pl.ds(base, _WIN)], vb[b], lvsem.at[b])

    def _wait_load_idx(w, b):
        base = in_base + w * _WIN
        pltpu.make_async_copy(idx_hbm.at[pl.ds(base, _WIN)], ib[b],
                              lisem.at[b]).wait()

    def _wait_load_val(w, b):
        base = in_base + w * _WIN
        pltpu.make_async_copy(upd_hbm.at[pl.ds(base, _WIN)], vb[b],
                              lvsem.at[b]).wait()

    def _filter(b, lo):
        @plsc.parallel_loop(0, _VSTEP, step=1, unroll=_UNROLL)
        def _vec(j):
            sl = pl.ds(j * _L, _L)
            x = ib[b][sl]
            u = plsc.bitcast(x - lo, jnp.uint32)
            d = plsc.bitcast((x & (_DUMMY - 1)) + _CH, jnp.uint32)
            ib[b][sl] = plsc.bitcast(jnp.minimum(u, d), jnp.int32)

    def _issue_scatter(b):
        pltpu.async_copy(vb[b], acc.at[ib[b]], scsem.at[b], add=True)

    def _wait_scatter(b):
        pltpu.make_async_copy(vb[b], acc.at[ib[b]], scsem.at[b]).wait()

    def _chunk(k, carry):
        lo = (cid * _CPS + k) * _CH
        zbase = pl.multiple_of(sid * _TSLICE, 8)

        pltpu.async_copy(zero_hbm.at[pl.ds(zbase, _TSLICE)],
                         acc.at[pl.ds(zbase, _TSLICE)], zsem)
        _issue_load(0, 0)
        pltpu.make_async_copy(zero_hbm.at[pl.ds(zbase, _TSLICE)],
                              acc.at[pl.ds(zbase, _TSLICE)], zsem).wait()
        plsc.subcore_barrier()

        def _group(g, c2):
            for b in range(_NB):
                w = g * _NB + b
                pf = (b + 1) % _NB
                if b < _NB - 1:
                    @pl.when(g > 0)
                    def _():
                        _wait_scatter(pf)
                    _issue_load(w + 1, pf)
                else:
                    _wait_scatter(pf)
                    @pl.when(g < _NWIN // _NB - 1)
                    def _():
                        _issue_load(w + 1, pf)
                _wait_load_idx(w, b)
                _filter(b, lo)
                _wait_load_val(w, b)
                _issue_scatter(b)
            return c2
        lax.fori_loop(0, _NWIN // _NB, _group, 0)

        _wait_scatter(1)
        _wait_scatter(2)
        plsc.subcore_barrier()

        off = pl.multiple_of(lo + sid * _TSLICE, 8)
        is_last = (cid * _CPS + k == _NCHUNK - 1) & (sid == _NT - 1)
        @pl.when(is_last)
        def _():
            pltpu.sync_copy(acc.at[pl.ds(zbase, _TCLIP)],
                            out_hbm.at[pl.ds(off, _TCLIP)])
        @pl.when(jnp.logical_not(is_last))
        def _():
            pltpu.sync_copy(acc.at[pl.ds(zbase, _TSLICE)],
                            out_hbm.at[pl.ds(off, _TSLICE)])
        return carry

    lax.fori_loop(0, _CPS, _chunk, 0)


def kernel(updates, mask):
    idx = mask.reshape(-1)
    upd = updates.reshape(-1)
    f = pl.kernel(
        _body,
        out_type=jax.ShapeDtypeStruct((_OUT,), jnp.float32),
        mesh=plsc.VectorSubcoreMesh(core_axis_name="c", subcore_axis_name="s"),
        scratch_types=[
            pltpu.VMEM((_WIN,), jnp.int32),
            pltpu.VMEM((_WIN,), jnp.int32),
            pltpu.VMEM((_WIN,), jnp.int32),
            pltpu.VMEM((_WIN,), jnp.float32),
            pltpu.VMEM((_WIN,), jnp.float32),
            pltpu.VMEM((_WIN,), jnp.float32),
            pltpu.VMEM_SHARED((_ACC,), jnp.float32),
            pltpu.SemaphoreType.DMA((_NB,)),
            pltpu.SemaphoreType.DMA((_NB,)),
            pltpu.SemaphoreType.DMA((_NB,)),
            pltpu.SemaphoreType.DMA,
        ],
    )
    zeros = jnp.zeros((_CH,), jnp.float32)
    out = f(idx, upd, zeros)
    return out.reshape(_B, _H * 2, _W * 2, _C)
